# baseline (speedup 1.0000x reference)
import functools

import jax
import jax.numpy as jnp
from jax import lax
from jax.experimental import pallas as pl
from jax.experimental.pallas import tpu as pltpu
from jax.experimental.pallas import tpu_sc as plsc

_TEMP = 0.1
_EPS = 1e-16
_BLK = 8000

_E = 320000
_NT = 16
_C = _E // _NT
_V = _C // 16
_NB = 1024
_NBG = _NB // 16


def _mlp_body(ea_ref, W1_ref, b1_ref, W2_ref, b2_ref, pi_ref, bmax_ref):
    ea = ea_ref[...]
    h = jnp.maximum(
        jnp.dot(ea, W1_ref[...], preferred_element_type=jnp.float32) + b1_ref[...],
        0.0,
    )
    pi = jnp.dot(h, W2_ref[...], preferred_element_type=jnp.float32) + b2_ref[...]
    pi_ref[...] = pi.reshape(1, 1, _BLK)
    bmax_ref[...] = jnp.max(pi).reshape(1, 1, 1)


def _exp_body(pi_ref, bmax_ref, e_ref, macc_ref):
    j = pl.program_id(0)

    @pl.when(j == 0)
    def _():
        macc_ref[0] = jnp.max(bmax_ref[...])

    ml = macc_ref[0] / _TEMP
    e_ref[...] = jnp.exp(pi_ref[...] / _TEMP - ml)


def _key_body(e_ref, den_ref, key_ref):
    score = jnp.maximum(e_ref[...] / (den_ref[0, 0] + _EPS), 0.0)
    bits = jax.lax.bitcast_convert_type(score, jnp.int32)
    key_ref[...] = 0x3F800000 - bits


def _sort_body(keys_hbm, perm_hbm, spKa, spIa, spH,
               kv, iv, posb, h2, tot, pfx, rowb, sem0, sem1):
    wid = lax.axis_index("s")
    lane = lax.iota(jnp.int32, 16)
    ones = jnp.ones((16,), jnp.int32)
    zeros = jnp.zeros((16,), jnp.int32)
    base = wid * _C

    def zero_h2():
        def zb(i, c):
            h2[pl.ds(i * 16, 16)] = zeros
            return c
        lax.fori_loop(0, _NB, zb, 0)

    def histogram(shift):
        def hb(i, c):
            idxv = lane * _V + i
            k = plsc.load_gather(kv, [idxv])
            d = lax.shift_right_logical(k, shift) & (_NB - 1)
            plsc.addupdate_scatter(h2, [d * 16 + lane], ones)
            return c
        lax.fori_loop(0, _V, hb, 0)

    def publish_totals():
        def tb(g, c):
            acc = zeros
            dsplat = (g * 16 + lane) * 16
            for l in range(16):
                acc = acc + plsc.load_gather(h2, [dsplat + l])
            tot[pl.ds(g * 16, 16)] = acc
            return c
        lax.fori_loop(0, _NBG, tb, 0)
        pltpu.sync_copy(tot, spH.at[pl.ds(wid * _NB, _NB)])

    def compute_bases():
        def zg(g, c):
            tot[pl.ds(g * 16, 16)] = zeros
            pfx[pl.ds(g * 16, 16)] = zeros
            return c
        lax.fori_loop(0, _NBG, zg, 0)
        for t in range(_NT):
            pltpu.sync_copy(spH.at[pl.ds(t * _NB, _NB)], rowb)

            def ab(g, c, _t=t):
                v = rowb[pl.ds(g * 16, 16)]
                tot[pl.ds(g * 16, 16)] = tot[pl.ds(g * 16, 16)] + v
                pfx[pl.ds(g * 16, 16)] = (
                    pfx[pl.ds(g * 16, 16)]
                    + v * jnp.where(jnp.int32(_t) < wid, 1, 0)
                )
                return c
            lax.fori_loop(0, _NBG, ab, 0)

        def sbod(g, carry):
            v = tot[pl.ds(g * 16, 16)]
            inc = plsc.cumsum(v)
            exc = inc - v + carry
            tot[pl.ds(g * 16, 16)] = exc + pfx[pl.ds(g * 16, 16)]
            return carry + jnp.sum(v)
        lax.fori_loop(0, _NBG, sbod, jnp.int32(0))

        def cbod(d, c):
            row = h2[pl.ds(d * 16, 16)]
            pre = plsc.cumsum(row) - row
            b = plsc.load_gather(tot, [zeros + d])
            h2[pl.ds(d * 16, 16)] = pre + b
            return c
        lax.fori_loop(0, _NB, cbod, 0)

    def permute(shift):
        def pb(i, c):
            idxv = lane * _V + i
            k = plsc.load_gather(kv, [idxv])
            d = lax.shift_right_logical(k, shift) & (_NB - 1)
            a = d * 16 + lane
            cc = plsc.load_gather(h2, [a])
            plsc.store_scatter(h2, [a], cc + ones)
            plsc.store_scatter(posb, [idxv], cc)
            return c
        lax.fori_loop(0, _V, pb, 0)

    def radix_pass(shift, src_k, src_i, dst_k, dst_i):
        if src_k is None:
            pltpu.sync_copy(keys_hbm.at[pl.ds(base, _C)], kv)

            def ib(i, c):
                iv[pl.ds(i * 16, 16)] = base + i * 16 + lane
                return c
            lax.fori_loop(0, _C // 16, ib, 0)
        else:
            pltpu.sync_copy(src_k.at[pl.ds(base, _C)], kv)
            pltpu.sync_copy(src_i.at[pl.ds(base, _C)], iv)
        zero_h2()
        histogram(shift)
        publish_totals()
        plsc.subcore_barrier()
        compute_bases()
        permute(shift)
        cpi = pltpu.async_copy(iv, dst_i.at[posb], sem1)
        if dst_k is not None:
            cpk = pltpu.async_copy(kv, dst_k.at[posb], sem0)
            cpk.wait()
        cpi.wait()
        plsc.subcore_barrier()

    radix_pass(0, None, None, spKa, spIa)
    radix_pass(10, spKa, spIa, spKa, spIa)
    radix_pass(20, spKa, spIa, None, spIa)
    pltpu.sync_copy(spIa.at[pl.ds(base, _C)], kv)
    pltpu.sync_copy(kv, perm_hbm.at[pl.ds(base, _C)])


_sort_kernel = functools.partial(
    pl.kernel,
    out_type=jax.ShapeDtypeStruct((_E,), jnp.int32),
    mesh=plsc.VectorSubcoreMesh(
        core_axis_name="c", subcore_axis_name="s", num_cores=1
    ),
    scratch_types=[
        pltpu.VMEM_SHARED((_E,), jnp.int32),
        pltpu.VMEM_SHARED((_E,), jnp.int32),
        pltpu.VMEM_SHARED((_NT * _NB,), jnp.int32),
        pltpu.VMEM((_C,), jnp.int32),
        pltpu.VMEM((_C,), jnp.int32),
        pltpu.VMEM((_C,), jnp.int32),
        pltpu.VMEM((_NB * 16,), jnp.int32),
        pltpu.VMEM((_NB,), jnp.int32),
        pltpu.VMEM((_NB,), jnp.int32),
        pltpu.VMEM((_NB,), jnp.int32),
        pltpu.SemaphoreType.DMA,
        pltpu.SemaphoreType.DMA,
    ],
    compiler_params=pltpu.CompilerParams(needs_layout_passes=False),
)(_sort_body)


_K = _E // 2
_CK = _K // _NT
_N = 10000


def _gather_body(perm_hbm, ei0_hbm, ei1_hbm, ea0_hbm, ea1_hbm,
                 ei_out, ea0_out, ea1_out,
                 spU, pcv, g0v, g1v, ga0v, ga1v, nv, onesv, sem0, sem1, sem2, sem3):
    wid = lax.axis_index("s")
    lane = lax.iota(jnp.int32, 16)
    ones = jnp.ones((16,), jnp.int32)
    zeros = jnp.zeros((16,), jnp.int32)
    base = wid * _CK

    pltpu.sync_copy(perm_hbm.at[pl.ds(base, _CK)], pcv)
    c0 = pltpu.async_copy(ei0_hbm.at[pcv], g0v, sem0)
    c1 = pltpu.async_copy(ei1_hbm.at[pcv], g1v, sem1)
    c2 = pltpu.async_copy(ea0_hbm.at[pcv], ga0v, sem2)
    c3 = pltpu.async_copy(ea1_hbm.at[pcv], ga1v, sem3)

    def ob(i, c):
        onesv[pl.ds(i * 16, 16)] = ones
        return c
    lax.fori_loop(0, _CK // 16, ob, 0)

    @pl.when(wid == 0)
    def _():
        def zb(i, c):
            nv[pl.ds(i * 16, 16)] = zeros
            return c
        lax.fori_loop(0, _N // 16, zb, 0)
        pltpu.sync_copy(nv, spU)

    c0.wait()
    c1.wait()
    plsc.subcore_barrier()

    pltpu.async_copy(onesv, spU.at[g0v], sem0, add=True).wait()
    pltpu.async_copy(onesv, spU.at[g1v], sem1, add=True).wait()
    plsc.subcore_barrier()

    @pl.when(wid == 0)
    def _():
        pltpu.sync_copy(spU, nv)

        def cb(g, carry):
            v = nv[pl.ds(g * 16, 16)]
            ind = jnp.where(v > 0, 1, 0)
            inc = plsc.cumsum(ind)
            nv[pl.ds(g * 16, 16)] = inc + carry - 1
            return carry + jnp.sum(ind)
        lax.fori_loop(0, _N // 16, cb, jnp.int32(0))
        pltpu.sync_copy(nv, spU)
    plsc.subcore_barrier()

    pltpu.sync_copy(spU, nv)

    def mb(i, c):
        v0 = g0v[pl.ds(i * 16, 16)]
        g0v[pl.ds(i * 16, 16)] = plsc.load_gather(nv, [v0])
        v1 = g1v[pl.ds(i * 16, 16)]
        g1v[pl.ds(i * 16, 16)] = plsc.load_gather(nv, [v1])
        return c
    lax.fori_loop(0, _CK // 16, mb, 0)

    c2.wait()
    c3.wait()
    pltpu.sync_copy(g0v, ei_out.at[pl.ds(base, _CK)])
    pltpu.sync_copy(g1v, ei_out.at[pl.ds(_K + base, _CK)])
    pltpu.sync_copy(ga0v, ea0_out.at[pl.ds(base, _CK)])
    pltpu.sync_copy(ga1v, ea1_out.at[pl.ds(base, _CK)])


_gather_kernel = functools.partial(
    pl.kernel,
    out_type=(
        jax.ShapeDtypeStruct((2 * _K,), jnp.int32),
        jax.ShapeDtypeStruct((_K,), jnp.float32),
        jax.ShapeDtypeStruct((_K,), jnp.float32),
    ),
    mesh=plsc.VectorSubcoreMesh(
        core_axis_name="c", subcore_axis_name="s", num_cores=1
    ),
    scratch_types=[
        pltpu.VMEM_SHARED((_N,), jnp.int32),
        pltpu.VMEM((_CK,), jnp.int32),
        pltpu.VMEM((_CK,), jnp.int32),
        pltpu.VMEM((_CK,), jnp.int32),
        pltpu.VMEM((_CK,), jnp.float32),
        pltpu.VMEM((_CK,), jnp.float32),
        pltpu.VMEM((_N,), jnp.int32),
        pltpu.VMEM((_CK,), jnp.int32),
        pltpu.SemaphoreType.DMA,
        pltpu.SemaphoreType.DMA,
        pltpu.SemaphoreType.DMA,
        pltpu.SemaphoreType.DMA,
    ],
    compiler_params=pltpu.CompilerParams(needs_layout_passes=False),
)(_gather_body)


def kernel(x, edge_index, edge_attr, batch, edge_batch, att, W1, b1, W2, b2):
    E = edge_attr.shape[0]
    grid = E // _BLK
    pi, bmax = pl.pallas_call(
        _mlp_body,
        grid=(grid,),
        in_specs=[
            pl.BlockSpec((_BLK, 2), lambda i: (i, 0)),
            pl.BlockSpec((2, 128), lambda i: (0, 0)),
            pl.BlockSpec((1, 128), lambda i: (0, 0)),
            pl.BlockSpec((128, 1), lambda i: (0, 0)),
            pl.BlockSpec((1, 1), lambda i: (0, 0)),
        ],
        out_specs=[
            pl.BlockSpec((1, 1, _BLK), lambda i: (i, 0, 0)),
            pl.BlockSpec((1, 1, 1), lambda i: (i, 0, 0)),
        ],
        out_shape=[
            jax.ShapeDtypeStruct((grid, 1, _BLK), jnp.float32),
            jax.ShapeDtypeStruct((grid, 1, 1), jnp.float32),
        ],
    )(edge_attr, W1, b1.reshape(1, 128), W2, b2.reshape(1, 1))

    e = pl.pallas_call(
        _exp_body,
        grid=(grid,),
        in_specs=[
            pl.BlockSpec((1, 1, _BLK), lambda i: (i, 0, 0)),
            pl.BlockSpec((grid, 1, 1), lambda i: (0, 0, 0)),
        ],
        out_specs=pl.BlockSpec((1, 1, _BLK), lambda i: (i, 0, 0)),
        out_shape=jax.ShapeDtypeStruct((grid, 1, _BLK), jnp.float32),
        scratch_shapes=[pltpu.SMEM((1,), jnp.float32)],
    )(pi, bmax)

    den = jax.ops.segment_sum(e.reshape(-1), edge_batch, num_segments=1).reshape(1, 1)

    keys = pl.pallas_call(
        _key_body,
        grid=(grid,),
        in_specs=[
            pl.BlockSpec((1, 1, _BLK), lambda i: (i, 0, 0)),
            pl.BlockSpec((1, 1), lambda i: (0, 0)),
        ],
        out_specs=pl.BlockSpec((1, 1, _BLK), lambda i: (i, 0, 0)),
        out_shape=jax.ShapeDtypeStruct((grid, 1, _BLK), jnp.int32),
    )(e, den)

    perm_full = _sort_kernel(keys.reshape(-1))
    ei_flat, eaA, eaB = _gather_kernel(
        perm_full, edge_index[0], edge_index[1],
        edge_attr[:, 0], edge_attr[:, 1],
    )
    ea2 = jnp.stack([eaA, eaB], axis=1)
    return (x, ei_flat.reshape(2, _K), ea2, batch)


# speedup vs baseline: 1.4077x; 1.4077x over previous
import functools

import jax
import jax.numpy as jnp
from jax import lax
from jax.experimental import pallas as pl
from jax.experimental.pallas import tpu as pltpu
from jax.experimental.pallas import tpu_sc as plsc

_TEMP = 0.1
_EPS = 1e-16
_BLK = 8000

_E = 320000
_NT = 16
_C = _E // _NT
_V = _C // 16
_NB = 1024
_NBG = _NB // 16


def _mlp_body(ea_ref, W1_ref, b1_ref, W2_ref, b2_ref, pi_ref, bmax_ref):
    ea = ea_ref[...]
    h = jnp.maximum(
        jnp.dot(ea, W1_ref[...], preferred_element_type=jnp.float32) + b1_ref[...],
        0.0,
    )
    pi = jnp.dot(h, W2_ref[...], preferred_element_type=jnp.float32) + b2_ref[...]
    pi_ref[...] = pi.reshape(1, 1, _BLK)
    bmax_ref[...] = jnp.max(pi).reshape(1, 1, 1)


def _exp_body(pi_ref, bmax_ref, e_ref, macc_ref):
    j = pl.program_id(0)

    @pl.when(j == 0)
    def _():
        macc_ref[0] = jnp.max(bmax_ref[...])

    ml = macc_ref[0] / _TEMP
    e_ref[...] = jnp.exp(pi_ref[...] / _TEMP - ml)


def _key_body(e_ref, den_ref, key_ref):
    score = jnp.maximum(e_ref[...] / (den_ref[0, 0] + _EPS), 0.0)
    bits = jax.lax.bitcast_convert_type(score, jnp.int32)
    key_ref[...] = 0x3F800000 - bits


def _sort_body(keys_hbm, perm_hbm, spKa, spIa, spH,
               kv, iv, posb, h2, tot, pfx, rowb, sem0, sem1):
    wid = lax.axis_index("s")
    lane = lax.iota(jnp.int32, 16)
    ones = jnp.ones((16,), jnp.int32)
    zeros = jnp.zeros((16,), jnp.int32)
    base = wid * _C

    def zero_h2():
        def zb(i, c):
            h2[pl.ds(i * 16, 16)] = zeros
            return c
        lax.fori_loop(0, _NB, zb, 0)

    def histogram(shift):
        def hb(i, c):
            idxv = lane * _V + i
            k = plsc.load_gather(kv, [idxv])
            d = lax.shift_right_logical(k, shift) & (_NB - 1)
            plsc.addupdate_scatter(h2, [d * 16 + lane], ones)
            return c
        lax.fori_loop(0, _V, hb, 0)

    def publish_totals():
        def tb(g, c):
            acc = zeros
            dsplat = (g * 16 + lane) * 16
            for l in range(16):
                acc = acc + plsc.load_gather(h2, [dsplat + l])
            tot[pl.ds(g * 16, 16)] = acc
            return c
        lax.fori_loop(0, _NBG, tb, 0)
        pltpu.sync_copy(tot, spH.at[pl.ds(wid * _NB, _NB)])

    def compute_bases():
        def zg(g, c):
            tot[pl.ds(g * 16, 16)] = zeros
            pfx[pl.ds(g * 16, 16)] = zeros
            return c
        lax.fori_loop(0, _NBG, zg, 0)
        for t in range(_NT):
            pltpu.sync_copy(spH.at[pl.ds(t * _NB, _NB)], rowb)

            def ab(g, c, _t=t):
                v = rowb[pl.ds(g * 16, 16)]
                tot[pl.ds(g * 16, 16)] = tot[pl.ds(g * 16, 16)] + v
                pfx[pl.ds(g * 16, 16)] = (
                    pfx[pl.ds(g * 16, 16)]
                    + v * jnp.where(jnp.int32(_t) < wid, 1, 0)
                )
                return c
            lax.fori_loop(0, _NBG, ab, 0)

        def sbod(g, carry):
            v = tot[pl.ds(g * 16, 16)]
            inc = plsc.cumsum(v)
            exc = inc - v + carry
            tot[pl.ds(g * 16, 16)] = exc + pfx[pl.ds(g * 16, 16)]
            return carry + jnp.sum(v)
        lax.fori_loop(0, _NBG, sbod, jnp.int32(0))

        def cbod(d, c):
            row = h2[pl.ds(d * 16, 16)]
            pre = plsc.cumsum(row) - row
            b = plsc.load_gather(tot, [zeros + d])
            h2[pl.ds(d * 16, 16)] = pre + b
            return c
        lax.fori_loop(0, _NB, cbod, 0)

    def permute(shift):
        def pb(i, c):
            idxv = lane * _V + i
            k = plsc.load_gather(kv, [idxv])
            d = lax.shift_right_logical(k, shift) & (_NB - 1)
            a = d * 16 + lane
            cc = plsc.load_gather(h2, [a])
            plsc.store_scatter(h2, [a], cc + ones)
            plsc.store_scatter(posb, [idxv], cc)
            return c
        lax.fori_loop(0, _V, pb, 0)

    def radix_pass(shift, src_k, src_i, dst_k, dst_i):
        if src_k is None:
            pltpu.sync_copy(keys_hbm.at[pl.ds(base, _C)], kv)

            def ib(i, c):
                iv[pl.ds(i * 16, 16)] = base + i * 16 + lane
                return c
            lax.fori_loop(0, _C // 16, ib, 0)
        else:
            pltpu.sync_copy(src_k.at[pl.ds(base, _C)], kv)
            pltpu.sync_copy(src_i.at[pl.ds(base, _C)], iv)
        zero_h2()
        histogram(shift)
        publish_totals()
        plsc.subcore_barrier()
        compute_bases()
        permute(shift)
        cpi = pltpu.async_copy(iv, dst_i.at[posb], sem1)
        if dst_k is not None:
            cpk = pltpu.async_copy(kv, dst_k.at[posb], sem0)
            cpk.wait()
        cpi.wait()
        plsc.subcore_barrier()

    radix_pass(0, None, None, spKa, spIa)
    radix_pass(10, spKa, spIa, spKa, spIa)
    radix_pass(20, spKa, spIa, None, spIa)
    pltpu.sync_copy(spIa.at[pl.ds(base, _C)], kv)
    pltpu.sync_copy(kv, perm_hbm.at[pl.ds(base, _C)])


_sort_kernel = functools.partial(
    pl.kernel,
    out_type=jax.ShapeDtypeStruct((_E,), jnp.int32),
    mesh=plsc.VectorSubcoreMesh(
        core_axis_name="c", subcore_axis_name="s", num_cores=1
    ),
    scratch_types=[
        pltpu.VMEM_SHARED((_E,), jnp.int32),
        pltpu.VMEM_SHARED((_E,), jnp.int32),
        pltpu.VMEM_SHARED((_NT * _NB,), jnp.int32),
        pltpu.VMEM((_C,), jnp.int32),
        pltpu.VMEM((_C,), jnp.int32),
        pltpu.VMEM((_C,), jnp.int32),
        pltpu.VMEM((_NB * 16,), jnp.int32),
        pltpu.VMEM((_NB,), jnp.int32),
        pltpu.VMEM((_NB,), jnp.int32),
        pltpu.VMEM((_NB,), jnp.int32),
        pltpu.SemaphoreType.DMA,
        pltpu.SemaphoreType.DMA,
    ],
    compiler_params=pltpu.CompilerParams(needs_layout_passes=False),
)(_sort_body)


_K = _E // 2
_CK = _K // _NT
_N = 10000


def _gather_body(perm_hbm, ei0_hbm, ei1_hbm, ea0_hbm, ea1_hbm,
                 ei_out, ea0_out, ea1_out,
                 spU, pcv, g0v, g1v, ga0v, ga1v, nv, onesv, sem0, sem1, sem2, sem3):
    wid = lax.axis_index("s")
    lane = lax.iota(jnp.int32, 16)
    ones = jnp.ones((16,), jnp.int32)
    zeros = jnp.zeros((16,), jnp.int32)
    base = wid * _CK

    pltpu.sync_copy(perm_hbm.at[pl.ds(base, _CK)], pcv)
    c0 = pltpu.async_copy(ei0_hbm.at[pcv], g0v, sem0)
    c1 = pltpu.async_copy(ei1_hbm.at[pcv], g1v, sem1)
    c2 = pltpu.async_copy(ea0_hbm.at[pcv], ga0v, sem2)
    c3 = pltpu.async_copy(ea1_hbm.at[pcv], ga1v, sem3)

    def ob(i, c):
        onesv[pl.ds(i * 16, 16)] = ones
        return c
    lax.fori_loop(0, _CK // 16, ob, 0)

    @pl.when(wid == 0)
    def _():
        def zb(i, c):
            nv[pl.ds(i * 16, 16)] = zeros
            return c
        lax.fori_loop(0, _N // 16, zb, 0)
        pltpu.sync_copy(nv, spU)

    c0.wait()
    c1.wait()
    plsc.subcore_barrier()

    pltpu.async_copy(onesv, spU.at[g0v], sem0, add=True).wait()
    pltpu.async_copy(onesv, spU.at[g1v], sem1, add=True).wait()
    plsc.subcore_barrier()

    @pl.when(wid == 0)
    def _():
        pltpu.sync_copy(spU, nv)

        def cb(g, carry):
            v = nv[pl.ds(g * 16, 16)]
            ind = jnp.where(v > 0, 1, 0)
            inc = plsc.cumsum(ind)
            nv[pl.ds(g * 16, 16)] = inc + carry - 1
            return carry + jnp.sum(ind)
        lax.fori_loop(0, _N // 16, cb, jnp.int32(0))
        pltpu.sync_copy(nv, spU)
    plsc.subcore_barrier()

    pltpu.sync_copy(spU, nv)

    def mb(i, c):
        v0 = g0v[pl.ds(i * 16, 16)]
        g0v[pl.ds(i * 16, 16)] = plsc.load_gather(nv, [v0])
        v1 = g1v[pl.ds(i * 16, 16)]
        g1v[pl.ds(i * 16, 16)] = plsc.load_gather(nv, [v1])
        return c
    lax.fori_loop(0, _CK // 16, mb, 0)

    c2.wait()
    c3.wait()
    pltpu.sync_copy(g0v, ei_out.at[pl.ds(base, _CK)])
    pltpu.sync_copy(g1v, ei_out.at[pl.ds(_K + base, _CK)])
    pltpu.sync_copy(ga0v, ea0_out.at[pl.ds(base, _CK)])
    pltpu.sync_copy(ga1v, ea1_out.at[pl.ds(base, _CK)])


_gather_kernel = functools.partial(
    pl.kernel,
    out_type=(
        jax.ShapeDtypeStruct((2 * _K,), jnp.int32),
        jax.ShapeDtypeStruct((_K,), jnp.float32),
        jax.ShapeDtypeStruct((_K,), jnp.float32),
    ),
    mesh=plsc.VectorSubcoreMesh(
        core_axis_name="c", subcore_axis_name="s", num_cores=1
    ),
    scratch_types=[
        pltpu.VMEM_SHARED((_N,), jnp.int32),
        pltpu.VMEM((_CK,), jnp.int32),
        pltpu.VMEM((_CK,), jnp.int32),
        pltpu.VMEM((_CK,), jnp.int32),
        pltpu.VMEM((_CK,), jnp.float32),
        pltpu.VMEM((_CK,), jnp.float32),
        pltpu.VMEM((_N,), jnp.int32),
        pltpu.VMEM((_CK,), jnp.int32),
        pltpu.SemaphoreType.DMA,
        pltpu.SemaphoreType.DMA,
        pltpu.SemaphoreType.DMA,
        pltpu.SemaphoreType.DMA,
    ],
    compiler_params=pltpu.CompilerParams(needs_layout_passes=False),
)(_gather_body)


def kernel(x, edge_index, edge_attr, batch, edge_batch, att, W1, b1, W2, b2):
    E = edge_attr.shape[0]
    grid = E // _BLK
    pi, bmax = pl.pallas_call(
        _mlp_body,
        grid=(grid,),
        in_specs=[
            pl.BlockSpec((_BLK, 2), lambda i: (i, 0)),
            pl.BlockSpec((2, 128), lambda i: (0, 0)),
            pl.BlockSpec((1, 128), lambda i: (0, 0)),
            pl.BlockSpec((128, 1), lambda i: (0, 0)),
            pl.BlockSpec((1, 1), lambda i: (0, 0)),
        ],
        out_specs=[
            pl.BlockSpec((1, 1, _BLK), lambda i: (i, 0, 0)),
            pl.BlockSpec((1, 1, 1), lambda i: (i, 0, 0)),
        ],
        out_shape=[
            jax.ShapeDtypeStruct((grid, 1, _BLK), jnp.float32),
            jax.ShapeDtypeStruct((grid, 1, 1), jnp.float32),
        ],
    )(edge_attr, W1, b1.reshape(1, 128), W2, b2.reshape(1, 1))

    e = pl.pallas_call(
        _exp_body,
        grid=(grid,),
        in_specs=[
            pl.BlockSpec((1, 1, _BLK), lambda i: (i, 0, 0)),
            pl.BlockSpec((grid, 1, 1), lambda i: (0, 0, 0)),
        ],
        out_specs=pl.BlockSpec((1, 1, _BLK), lambda i: (i, 0, 0)),
        out_shape=jax.ShapeDtypeStruct((grid, 1, _BLK), jnp.float32),
        scratch_shapes=[pltpu.SMEM((1,), jnp.float32)],
    )(pi, bmax)

    den = jax.ops.segment_sum(
        e.reshape(-1), edge_batch, num_segments=1, indices_are_sorted=True
    ).reshape(1, 1)

    keys = pl.pallas_call(
        _key_body,
        grid=(grid,),
        in_specs=[
            pl.BlockSpec((1, 1, _BLK), lambda i: (i, 0, 0)),
            pl.BlockSpec((1, 1), lambda i: (0, 0)),
        ],
        out_specs=pl.BlockSpec((1, 1, _BLK), lambda i: (i, 0, 0)),
        out_shape=jax.ShapeDtypeStruct((grid, 1, _BLK), jnp.int32),
    )(e, den)

    perm_full = _sort_kernel(keys.reshape(-1))
    ei_flat, eaA, eaB = _gather_kernel(
        perm_full, edge_index[0], edge_index[1],
        edge_attr[:, 0], edge_attr[:, 1],
    )
    ea2 = jnp.stack([eaA, eaB], axis=1)
    return (x, ei_flat.reshape(2, _K), ea2, batch)
